# baseline (speedup 1.0000x reference)
import functools

import jax
import jax.numpy as jnp
from jax import lax
from jax.experimental import pallas as pl
from jax.experimental.pallas import tpu as pltpu
from jax.experimental.pallas import tpu_sc as plsc

N = 10000
E = 320000
D_IN = 128

NC = 2
NS = 16
NW = NC * NS
CH = 128
KJ = 80
KJ0 = 112
KJ1 = 48
GS = 16
EPW = CH * KJ
E_PAD = EPW * NW
NSEG = 2
NP = 10112
SEG = NP // NSEG
NPH = 5120
RPS = NPH // NS

_mesh = plsc.VectorSubcoreMesh(core_axis_name="c", subcore_axis_name="s")


def _make_spmm(do_deg):
    scratch = [
        pltpu.VMEM((GS, CH), jnp.int32),
        pltpu.VMEM((GS, CH), jnp.int32),
        pltpu.VMEM((GS, CH), jnp.int32),
        pltpu.VMEM((GS, CH), jnp.int32),
        pltpu.VMEM((CH, D_IN), jnp.float32),
        pltpu.VMEM((CH, D_IN), jnp.float32),
        pltpu.SemaphoreType.DMA,
        pltpu.SemaphoreType.DMA,
        pltpu.SemaphoreType.DMA,
        pltpu.SemaphoreType.DMA,
        pltpu.SemaphoreType.DMA,
        pltpu.SemaphoreType.DMA,
    ] + [pltpu.VMEM_SHARED((NPH, D_IN), jnp.float32) for _ in range(NSEG)]

    @functools.partial(
        pl.kernel, mesh=_mesh,
        out_type=jax.ShapeDtypeStruct((NC * NSEG * NPH, D_IN), jnp.float32),
        scratch_types=scratch)
    def spmm(row_hbm, col_hbm, table_hbm, zseg_hbm, ones_hbm, out_hbm,
             row_v, col_v, idx0_v, idx1_v, rows_a, rows_b,
             semg_a, semg_b, sems0_a, sems0_b, sems1_a, sems1_b, *accs):
        cid = lax.axis_index("c")
        sid = lax.axis_index("s")
        g = cid * NS + sid
        r0 = sid * RPS
        bufs = (rows_a, rows_b)
        gsems = (semg_a, semg_b)
        ssems = ((sems0_a, sems0_b), (sems1_a, sems1_b))
        idxs = (idx0_v, idx1_v)

        for s in range(NSEG):
            pltpu.sync_copy(zseg_hbm.at[pl.ds(r0, RPS)],
                            accs[s].at[pl.ds(r0, RPS)])
        if do_deg:
            pltpu.sync_copy(ones_hbm, rows_a)
        plsc.subcore_barrier()

        def mask_group():
            def mj(j, c1):
                def m16(k, c2):
                    r = row_v[j, pl.ds(k * 16, 16)]
                    c = col_v[j, pl.ds(k * 16, 16)]
                    for s in range(NSEG):
                        lo = r - jnp.int32(s * SEG)
                        ok = (r != c) & (lo >= 0) & (lo < jnp.int32(SEG))
                        idxs[s][j, pl.ds(k * 16, 16)] = jnp.where(
                            ok, lo,
                            jnp.int32(SEG) + lax.iota(jnp.int32, 16))
                    return c2
                return lax.fori_loop(0, CH // 16, m16, c1)
            lax.fori_loop(0, GS, mj, 0)

        if do_deg:
            base = g * KJ
            ngrp = KJ // GS
        else:
            base = pl.multiple_of(
                jnp.where(cid == 0, sid * KJ0, NS * KJ0 + sid * KJ1), 8)
            ngrp = jnp.where(cid == 0, KJ0 // GS, KJ1 // GS)

        def group(b, carry):
            pltpu.sync_copy(row_hbm.at[pl.ds(base + b * GS, GS)], row_v)
            pltpu.sync_copy(col_hbm.at[pl.ds(base + b * GS, GS)], col_v)
            mask_group()
            if do_deg:
                sc = {}
                for j in range(GS):
                    if j >= 2:
                        sc[j - 2][0].wait()
                        sc[j - 2][1].wait()
                    sc[j] = tuple(
                        pltpu.async_copy(rows_a,
                                         accs[s].at[idxs[s].at[j]],
                                         ssems[s][j % 2], add=True)
                        for s in range(NSEG))
                for j in (GS - 2, GS - 1):
                    sc[j][0].wait()
                    sc[j][1].wait()
                return carry
            cps = {0: pltpu.async_copy(table_hbm.at[col_v.at[0]], rows_a,
                                       semg_a)}
            sc = {}
            for j in range(GS):
                cps[j].wait()
                sc[j] = tuple(
                    pltpu.async_copy(bufs[j % 2],
                                     accs[s].at[idxs[s].at[j]],
                                     ssems[s][j % 2], add=True)
                    for s in range(NSEG))
                if j < GS - 1:
                    if j >= 1:
                        sc[j - 1][0].wait()
                        sc[j - 1][1].wait()
                    cps[j + 1] = pltpu.async_copy(
                        table_hbm.at[col_v.at[j + 1]], bufs[(j + 1) % 2],
                        gsems[(j + 1) % 2])
            for j in (GS - 2, GS - 1):
                sc[j][0].wait()
                sc[j][1].wait()
            return carry
        lax.fori_loop(0, ngrp, group, 0)
        plsc.subcore_barrier()

        for s in range(NSEG):
            off = 0
            while off < RPS:
                sz = min(128, RPS - off)
                pltpu.sync_copy(
                    accs[s].at[pl.ds(r0 + off, sz)],
                    out_hbm.at[pl.ds((cid * NSEG + s) * NPH + r0 + off,
                                     sz)])
                off += sz

    return spmm


_spmm = _make_spmm(False)
_degcnt = _make_spmm(True)


def _matmul_h(x, wcat):
    def mm(x_ref, w_ref, o_ref):
        o_ref[...] = jnp.dot(x_ref[...], w_ref[...],
                             preferred_element_type=jnp.float32)
    return pl.pallas_call(
        mm,
        grid=(10,),
        in_specs=[pl.BlockSpec((N // 10, D_IN), lambda i: (i, 0)),
                  pl.BlockSpec((D_IN, D_IN), lambda i: (0, 0))],
        out_specs=pl.BlockSpec((N // 10, D_IN), lambda i: (i, 0)),
        out_shape=jax.ShapeDtypeStruct((N, D_IN), jnp.float32),
    )(x, wcat)


def _combine1(p0, p1, d0, d1, h, b):
    def body(p0_ref, p1_ref, d0_ref, d1_ref, h_ref, b_ref,
             out0_ref, y1_ref):
        deg = d0_ref[:, 0:1] + d1_ref[:, 0:1] + 1.0
        s = (p0_ref[...] + p1_ref[...] + h_ref[...]) / deg
        out0_ref[...] = jnp.maximum(s[:, :64] + b_ref[0:1, :64], 0.0)
        y1_ref[:, :64] = s[:, 64:]
        y1_ref[:, 64:] = jnp.zeros_like(s[:, 64:])
    bn = N // 10
    return pl.pallas_call(
        body,
        grid=(10,),
        in_specs=[pl.BlockSpec((bn, D_IN), lambda i: (i, 0)),
                  pl.BlockSpec((bn, D_IN), lambda i: (i, 0)),
                  pl.BlockSpec((bn, D_IN), lambda i: (i, 0)),
                  pl.BlockSpec((bn, D_IN), lambda i: (i, 0)),
                  pl.BlockSpec((bn, D_IN), lambda i: (i, 0)),
                  pl.BlockSpec((1, D_IN), lambda i: (0, 0))],
        out_specs=[pl.BlockSpec((bn, 64), lambda i: (i, 0)),
                   pl.BlockSpec((bn, D_IN), lambda i: (i, 0))],
        out_shape=[jax.ShapeDtypeStruct((N, 64), jnp.float32),
                   jax.ShapeDtypeStruct((N, D_IN), jnp.float32)],
    )(p0, p1, d0, d1, h, b)


def _combine2(q0, q1, d0, d1, y1, b):
    def body(q0_ref, q1_ref, d0_ref, d1_ref, y1_ref, b_ref, out1_ref):
        deg = d0_ref[:, 0:1] + d1_ref[:, 0:1] + 1.0
        s = (q0_ref[:, :64] + q1_ref[:, :64] + y1_ref[:, :64]) / deg
        out1_ref[...] = jnp.maximum(s + b_ref[0:1, 64:], 0.0)
    bn = N // 10
    return pl.pallas_call(
        body,
        grid=(10,),
        in_specs=[pl.BlockSpec((bn, D_IN), lambda i: (i, 0)),
                  pl.BlockSpec((bn, D_IN), lambda i: (i, 0)),
                  pl.BlockSpec((bn, D_IN), lambda i: (i, 0)),
                  pl.BlockSpec((bn, D_IN), lambda i: (i, 0)),
                  pl.BlockSpec((bn, D_IN), lambda i: (i, 0)),
                  pl.BlockSpec((1, D_IN), lambda i: (0, 0))],
        out_specs=pl.BlockSpec((bn, 64), lambda i: (i, 0)),
        out_shape=jax.ShapeDtypeStruct((N, 64), jnp.float32),
    )(q0, q1, d0, d1, y1, b)


def _per_core(out):
    parts = []
    for c in range(NC):
        segs = [out[(c * NSEG + s) * NPH:(c * NSEG + s) * NPH + SEG]
                for s in range(NSEG)]
        parts.append(jnp.concatenate(segs, axis=0)[:N])
    return parts


def kernel(x, edge_index, W0, W1, b0, b1, att_p, fc0, fc1, bf0, bf1):
    row = edge_index[0]
    col = edge_index[1]
    pad = jnp.zeros((E_PAD - E,), jnp.int32)
    row2 = jnp.concatenate([row, pad]).reshape(E_PAD // CH, CH)
    col2 = jnp.concatenate([col, pad]).reshape(E_PAD // CH, CH)

    wcat = jnp.concatenate([W0, W1], axis=1)
    h = _matmul_h(x, wcat)

    zseg = jnp.zeros((NPH, D_IN), jnp.float32)
    ones = jnp.ones((CH, D_IN), jnp.float32)

    d = _degcnt(row2, col2, h, zseg, ones)
    d0, d1 = _per_core(d)

    p = _spmm(row2, col2, h, zseg, ones)
    p0, p1 = _per_core(p)

    bcat = jnp.concatenate([b0, b1]).reshape(1, D_IN)
    out0, y1 = _combine1(p0, p1, d0, d1, h, bcat)

    q = _spmm(row2, col2, y1, zseg, ones)
    q0, q1 = _per_core(q)

    out1 = _combine2(q0, q1, d0, d1, y1, bcat)
    return jnp.concatenate([out0, out1], axis=1)


# speedup vs baseline: 1.0630x; 1.0630x over previous
import functools

import jax
import jax.numpy as jnp
from jax import lax
from jax.experimental import pallas as pl
from jax.experimental.pallas import tpu as pltpu
from jax.experimental.pallas import tpu_sc as plsc

N = 10000
E = 320000
D_IN = 128

NC = 2
NS = 16
NW = NC * NS
CH = 128
KJ = 80
KJ0 = 128
KJ1 = 32
EPW = CH * KJ
E_PAD = EPW * NW
NSEG = 2
NP = 10112
SEG = NP // NSEG
NPH = 5120
RPS = NPH // NS

_mesh = plsc.VectorSubcoreMesh(core_axis_name="c", subcore_axis_name="s")


def _make_spmm(do_deg):
    scratch = [
        pltpu.VMEM((8, CH), jnp.int32),
        pltpu.VMEM((8, CH), jnp.int32),
        pltpu.VMEM((8, CH), jnp.int32),
        pltpu.VMEM((8, CH), jnp.int32),
        pltpu.VMEM((CH, D_IN), jnp.float32),
        pltpu.VMEM((CH, D_IN), jnp.float32),
        pltpu.SemaphoreType.DMA,
        pltpu.SemaphoreType.DMA,
        pltpu.SemaphoreType.DMA,
        pltpu.SemaphoreType.DMA,
        pltpu.SemaphoreType.DMA,
        pltpu.SemaphoreType.DMA,
    ] + [pltpu.VMEM_SHARED((NPH, D_IN), jnp.float32) for _ in range(NSEG)]

    @functools.partial(
        pl.kernel, mesh=_mesh,
        out_type=jax.ShapeDtypeStruct((NC * NSEG * NPH, D_IN), jnp.float32),
        scratch_types=scratch)
    def spmm(row_hbm, col_hbm, table_hbm, zseg_hbm, ones_hbm, out_hbm,
             row_v, col_v, idx0_v, idx1_v, rows_a, rows_b,
             semg_a, semg_b, sems0_a, sems0_b, sems1_a, sems1_b, *accs):
        cid = lax.axis_index("c")
        sid = lax.axis_index("s")
        g = cid * NS + sid
        r0 = sid * RPS
        bufs = (rows_a, rows_b)
        gsems = (semg_a, semg_b)
        ssems = ((sems0_a, sems0_b), (sems1_a, sems1_b))
        idxs = (idx0_v, idx1_v)

        for s in range(NSEG):
            pltpu.sync_copy(zseg_hbm.at[pl.ds(r0, RPS)],
                            accs[s].at[pl.ds(r0, RPS)])
        if do_deg:
            pltpu.sync_copy(ones_hbm, rows_a)
        plsc.subcore_barrier()

        def mask_group():
            def mj(j, c1):
                def m16(k, c2):
                    r = row_v[j, pl.ds(k * 16, 16)]
                    c = col_v[j, pl.ds(k * 16, 16)]
                    for s in range(NSEG):
                        lo = r - jnp.int32(s * SEG)
                        ok = (r != c) & (lo >= 0) & (lo < jnp.int32(SEG))
                        idxs[s][j, pl.ds(k * 16, 16)] = jnp.where(
                            ok, lo,
                            jnp.int32(SEG) + lax.iota(jnp.int32, 16))
                    return c2
                return lax.fori_loop(0, CH // 16, m16, c1)
            lax.fori_loop(0, 8, mj, 0)

        if do_deg:
            base = g * KJ
            ngrp = KJ // 8
        else:
            base = pl.multiple_of(
                jnp.where(cid == 0, sid * KJ0, NS * KJ0 + sid * KJ1), 8)
            ngrp = jnp.where(cid == 0, KJ0 // 8, KJ1 // 8)

        def group(b, carry):
            pltpu.sync_copy(row_hbm.at[pl.ds(base + b * 8, 8)], row_v)
            pltpu.sync_copy(col_hbm.at[pl.ds(base + b * 8, 8)], col_v)
            mask_group()
            if do_deg:
                sc = {}
                for j in range(8):
                    if j >= 2:
                        sc[j - 2][0].wait()
                        sc[j - 2][1].wait()
                    sc[j] = tuple(
                        pltpu.async_copy(rows_a,
                                         accs[s].at[idxs[s].at[j]],
                                         ssems[s][j % 2], add=True)
                        for s in range(NSEG))
                for j in (6, 7):
                    sc[j][0].wait()
                    sc[j][1].wait()
                return carry
            cps = {0: pltpu.async_copy(table_hbm.at[col_v.at[0]], rows_a,
                                       semg_a)}
            sc = {}
            for j in range(8):
                cps[j].wait()
                sc[j] = tuple(
                    pltpu.async_copy(bufs[j % 2],
                                     accs[s].at[idxs[s].at[j]],
                                     ssems[s][j % 2], add=True)
                    for s in range(NSEG))
                if j < 7:
                    if j >= 1:
                        sc[j - 1][0].wait()
                        sc[j - 1][1].wait()
                    cps[j + 1] = pltpu.async_copy(
                        table_hbm.at[col_v.at[j + 1]], bufs[(j + 1) % 2],
                        gsems[(j + 1) % 2])
            for j in (6, 7):
                sc[j][0].wait()
                sc[j][1].wait()
            return carry
        lax.fori_loop(0, ngrp, group, 0)
        plsc.subcore_barrier()

        for s in range(NSEG):
            off = 0
            while off < RPS:
                sz = min(128, RPS - off)
                pltpu.sync_copy(
                    accs[s].at[pl.ds(r0 + off, sz)],
                    out_hbm.at[pl.ds((cid * NSEG + s) * NPH + r0 + off,
                                     sz)])
                off += sz

    return spmm


_spmm = _make_spmm(False)
_degcnt = _make_spmm(True)


def _matmul_h(x, wcat):
    def mm(x_ref, w_ref, o_ref):
        o_ref[...] = jnp.dot(x_ref[...], w_ref[...],
                             preferred_element_type=jnp.float32)
    return pl.pallas_call(
        mm,
        grid=(10,),
        in_specs=[pl.BlockSpec((N // 10, D_IN), lambda i: (i, 0)),
                  pl.BlockSpec((D_IN, D_IN), lambda i: (0, 0))],
        out_specs=pl.BlockSpec((N // 10, D_IN), lambda i: (i, 0)),
        out_shape=jax.ShapeDtypeStruct((N, D_IN), jnp.float32),
    )(x, wcat)


def _combine1(p0, p1, d0, d1, h, b):
    def body(p0_ref, p1_ref, d0_ref, d1_ref, h_ref, b_ref,
             out0_ref, y1_ref):
        deg = d0_ref[:, 0:1] + d1_ref[:, 0:1] + 1.0
        s = (p0_ref[...] + p1_ref[...] + h_ref[...]) / deg
        out0_ref[...] = jnp.maximum(s[:, :64] + b_ref[0:1, :64], 0.0)
        y1_ref[:, :64] = s[:, 64:]
        y1_ref[:, 64:] = jnp.zeros_like(s[:, 64:])
    bn = N // 10
    return pl.pallas_call(
        body,
        grid=(10,),
        in_specs=[pl.BlockSpec((bn, D_IN), lambda i: (i, 0)),
                  pl.BlockSpec((bn, D_IN), lambda i: (i, 0)),
                  pl.BlockSpec((bn, D_IN), lambda i: (i, 0)),
                  pl.BlockSpec((bn, D_IN), lambda i: (i, 0)),
                  pl.BlockSpec((bn, D_IN), lambda i: (i, 0)),
                  pl.BlockSpec((1, D_IN), lambda i: (0, 0))],
        out_specs=[pl.BlockSpec((bn, 64), lambda i: (i, 0)),
                   pl.BlockSpec((bn, D_IN), lambda i: (i, 0))],
        out_shape=[jax.ShapeDtypeStruct((N, 64), jnp.float32),
                   jax.ShapeDtypeStruct((N, D_IN), jnp.float32)],
    )(p0, p1, d0, d1, h, b)


def _combine2(q0, q1, d0, d1, y1, b):
    def body(q0_ref, q1_ref, d0_ref, d1_ref, y1_ref, b_ref, out1_ref):
        deg = d0_ref[:, 0:1] + d1_ref[:, 0:1] + 1.0
        s = (q0_ref[:, :64] + q1_ref[:, :64] + y1_ref[:, :64]) / deg
        out1_ref[...] = jnp.maximum(s + b_ref[0:1, 64:], 0.0)
    bn = N // 10
    return pl.pallas_call(
        body,
        grid=(10,),
        in_specs=[pl.BlockSpec((bn, D_IN), lambda i: (i, 0)),
                  pl.BlockSpec((bn, D_IN), lambda i: (i, 0)),
                  pl.BlockSpec((bn, D_IN), lambda i: (i, 0)),
                  pl.BlockSpec((bn, D_IN), lambda i: (i, 0)),
                  pl.BlockSpec((bn, D_IN), lambda i: (i, 0)),
                  pl.BlockSpec((1, D_IN), lambda i: (0, 0))],
        out_specs=pl.BlockSpec((bn, 64), lambda i: (i, 0)),
        out_shape=jax.ShapeDtypeStruct((N, 64), jnp.float32),
    )(q0, q1, d0, d1, y1, b)


def _per_core(out):
    parts = []
    for c in range(NC):
        segs = [out[(c * NSEG + s) * NPH:(c * NSEG + s) * NPH + SEG]
                for s in range(NSEG)]
        parts.append(jnp.concatenate(segs, axis=0)[:N])
    return parts


def kernel(x, edge_index, W0, W1, b0, b1, att_p, fc0, fc1, bf0, bf1):
    row = edge_index[0]
    col = edge_index[1]
    pad = jnp.zeros((E_PAD - E,), jnp.int32)
    row2 = jnp.concatenate([row, pad]).reshape(E_PAD // CH, CH)
    col2 = jnp.concatenate([col, pad]).reshape(E_PAD // CH, CH)

    wcat = jnp.concatenate([W0, W1], axis=1)
    h = _matmul_h(x, wcat)

    zseg = jnp.zeros((NPH, D_IN), jnp.float32)
    ones = jnp.ones((CH, D_IN), jnp.float32)

    d = _degcnt(row2, col2, h, zseg, ones)
    d0, d1 = _per_core(d)

    p = _spmm(row2, col2, h, zseg, ones)
    p0, p1 = _per_core(p)

    bcat = jnp.concatenate([b0, b1]).reshape(1, D_IN)
    out0, y1 = _combine1(p0, p1, d0, d1, h, bcat)

    q = _spmm(row2, col2, y1, zseg, ones)
    q0, q1 = _per_core(q)

    out1 = _combine2(q0, q1, d0, d1, y1, bcat)
    return jnp.concatenate([out0, out1], axis=1)
